# Initial kernel scaffold; baseline (speedup 1.0000x reference)
import jax
import jax.numpy as jnp
from jax.experimental import pallas as pl


def kernel(x, edge_index, W1l, b1l, W1r, W2l, b2l, W2r, Wh, bh):
    raise NotImplementedError("write your pallas kernel here")


# speedup vs baseline: 7.9056x; 7.9056x over previous
import functools

import jax
import jax.numpy as jnp
from jax import lax
from jax.experimental import pallas as pl
from jax.experimental.pallas import tpu as pltpu
from jax.experimental.pallas import tpu_sc as plsc

N = 10000
E = 320000
IN_DIM = 128
HID = 64

NC = 2
NS = 16
NW = NC * NS
L = 16

C = 128
K = -(-E // (NW * C))
EPW = K * C
E_PAD = NW * EPW
N_PAD = 10240
RPS = N_PAD // NS

f32 = jnp.float32


def _make_seg_sum(with_counts: bool):
    out_type = [jax.ShapeDtypeStruct((NC, N_PAD, HID), f32)]
    scratch = [
        pltpu.VMEM((K, C), jnp.int32),
        pltpu.VMEM((K, C), jnp.int32),
        pltpu.VMEM((C, HID), f32),
        pltpu.VMEM((C, HID), f32),
        pltpu.VMEM_SHARED((N_PAD, HID), f32),
        pltpu.SemaphoreType.DMA,
    ]
    if with_counts:
        out_type.append(jax.ShapeDtypeStruct((NC, N_PAD), f32))
        scratch += [
            pltpu.VMEM((C,), f32),
            pltpu.VMEM((RPS,), f32),
            pltpu.VMEM_SHARED((N_PAD,), f32),
        ]

    mesh = plsc.VectorSubcoreMesh(core_axis_name="c", subcore_axis_name="s")

    def body(table, srcs, dsts, *refs):
        if with_counts:
            (out_sums, out_cnts, src_v, dst_v, rows_v, zrow_v, acc_sh, sem,
             ones_v, zcnt_v, cnt_sh) = refs
        else:
            out_sums, src_v, dst_v, rows_v, zrow_v, acc_sh, sem = refs
        cid = lax.axis_index("c")
        sid = lax.axis_index("s")
        wid = sid * NC + cid

        zv = jnp.zeros((L,), f32)

        def zrow_body(i, _):
            for j in range(HID // L):
                zrow_v[i, pl.ds(j * L, L)] = zv
            return _

        lax.fori_loop(0, C, zrow_body, None)
        if with_counts:
            ov = jnp.ones((L,), f32)
            for j in range(C // L):
                ones_v[pl.ds(j * L, L)] = ov
            for j in range(RPS // L):
                zcnt_v[pl.ds(j * L, L)] = zv

        for b in range(RPS // C):
            pltpu.sync_copy(zrow_v, acc_sh.at[pl.ds(sid * RPS + b * C, C)])
        if with_counts:
            pltpu.sync_copy(zcnt_v, cnt_sh.at[pl.ds(sid * RPS, RPS)])
        plsc.subcore_barrier()

        pltpu.sync_copy(srcs.at[wid], src_v)
        pltpu.sync_copy(dsts.at[wid], dst_v)

        def chunk(k, _):
            pltpu.async_copy(table.at[src_v.at[k]], rows_v, sem).wait()
            pltpu.sync_copy(rows_v, acc_sh.at[dst_v.at[k]], add=True)
            if with_counts:
                pltpu.sync_copy(ones_v, cnt_sh.at[dst_v.at[k]], add=True)
            return _

        lax.fori_loop(0, K, chunk, None)
        plsc.subcore_barrier()

        rows = pl.ds(sid * RPS, RPS)
        pltpu.sync_copy(acc_sh.at[rows], out_sums.at[cid, rows])
        if with_counts:
            pltpu.sync_copy(cnt_sh.at[rows], out_cnts.at[cid, rows])

    return pl.kernel(body, out_type=tuple(out_type), mesh=mesh,
                     scratch_types=scratch,
                     compiler_params=pltpu.CompilerParams(
                         use_tc_tiling_on_sc=False))


_seg_sum_cnt = _make_seg_sum(with_counts=True)
_seg_sum = _make_seg_sum(with_counts=False)


_DN = (((1,), (1,)), ((), ()))


def _lin1_body(x_ref, wl_ref, wr_ref, xl_ref, xr_ref):
    x = x_ref[...]
    xl_ref[...] = lax.dot_general(x, wl_ref[...], _DN,
                                  preferred_element_type=f32)
    xr_ref[...] = lax.dot_general(x, wr_ref[...], _DN,
                                  preferred_element_type=f32)


def _mid_body(s_ref, c_ref, xr_ref, b1_ref, w2l_ref, w2r_ref,
              hl_ref, hr_ref):
    s = s_ref[0, ...] + s_ref[1, ...]
    c = jnp.maximum(c_ref[0, ...] + c_ref[1, ...], 1.0)
    h = jnp.maximum(s / c + b1_ref[...] + xr_ref[...], 0.0)
    hl_ref[...] = lax.dot_general(h, w2l_ref[...], _DN,
                                  preferred_element_type=f32)
    hr_ref[...] = lax.dot_general(h, w2r_ref[...], _DN,
                                  preferred_element_type=f32)


def _head_body(s_ref, c_ref, hr_ref, b2_ref, wh_ref, bh_ref, out_ref):
    s = s_ref[0, ...] + s_ref[1, ...]
    c = jnp.maximum(c_ref[0, ...] + c_ref[1, ...], 1.0)
    h2 = jnp.maximum(s / c + b2_ref[...] + hr_ref[...], 0.0)
    out_ref[...] = lax.dot_general(h2, wh_ref[...], _DN,
                                   preferred_element_type=f32) + bh_ref[0, 0]


def kernel(x, edge_index, W1l, b1l, W1r, W2l, b2l, W2r, Wh, bh):
    src = edge_index[0].astype(jnp.int32)
    dst = edge_index[1].astype(jnp.int32)
    pad = E_PAD - E
    src_p = jnp.concatenate([src, jnp.full((pad,), N, jnp.int32)])
    dst_p = jnp.concatenate([dst, jnp.full((pad,), N, jnp.int32)])
    srcs = src_p.reshape(NW, K, C)
    dsts = dst_p.reshape(NW, K, C)

    x_pad = jnp.concatenate([x, jnp.zeros((N_PAD - N, IN_DIM), f32)])

    xl, xr = pl.pallas_call(
        _lin1_body,
        out_shape=(jax.ShapeDtypeStruct((N_PAD, HID), f32),
                   jax.ShapeDtypeStruct((N_PAD, HID), f32)),
    )(x_pad, W1l, W1r)

    sums1, cnts = _seg_sum_cnt(xl, srcs, dsts)
    cnts3 = cnts.reshape(NC, N_PAD, 1)

    hl, hr = pl.pallas_call(
        _mid_body,
        out_shape=(jax.ShapeDtypeStruct((N_PAD, HID), f32),
                   jax.ShapeDtypeStruct((N_PAD, HID), f32)),
    )(sums1, cnts3, xr, b1l.reshape(1, HID), W2l, W2r)

    (sums2,) = _seg_sum(hl, srcs, dsts)

    Wh_p = jnp.concatenate([Wh, jnp.zeros((7, HID), f32)])
    logits = pl.pallas_call(
        _head_body,
        out_shape=jax.ShapeDtypeStruct((N_PAD, 8), f32),
    )(sums2, cnts3, hr, b2l.reshape(1, HID), Wh_p, bh.reshape(1, 1))

    return logits[:N, 0]
